# baseline (speedup 1.0000x reference)
import functools

import jax
import jax.numpy as jnp
from jax import lax
from jax.experimental import pallas as pl
from jax.experimental.pallas import tpu as pltpu
from jax.experimental.pallas import tpu_sc as plsc

NUM_INPUTS = 4096
ADDRESS_SIZE = 4
NUM_LUTS = NUM_INPUTS // ADDRESS_SIZE
ENTRY_PER_LUT = 2 ** ADDRESS_SIZE
BATCH = 4096

_INFO = plsc.get_sparse_core_info()
_NC = _INFO.num_cores
_NS = _INFO.num_subcores
_L = _INFO.num_lanes
_NW = _NC * _NS
_ROWS_PER_W = BATCH // _NW
_G = _L
_NGROUPS = _ROWS_PER_W // _G


def _make_kernel():
    mesh = plsc.VectorSubcoreMesh(core_axis_name="c", subcore_axis_name="s")

    @functools.partial(
        pl.kernel,
        mesh=mesh,
        compiler_params=pltpu.CompilerParams(needs_layout_passes=False),
        out_type=jax.ShapeDtypeStruct((BATCH,), jnp.float32),
        scratch_types=[
            pltpu.VMEM((NUM_LUTS * ENTRY_PER_LUT,), jnp.float32),
            pltpu.VMEM((_G * NUM_INPUTS,), jnp.int32),
            pltpu.VMEM((_G,), jnp.float32),
        ],
    )
    def k(x_hbm, luts_hbm, out_hbm, luts_v, xg_v, out_v):
        wid = lax.axis_index("s") * _NC + lax.axis_index("c")
        pltpu.sync_copy(luts_hbm, luts_v)
        lane = lax.broadcasted_iota(jnp.int32, (_L,), 0)
        row_off = lane * NUM_INPUTS

        for g in range(_NGROUPS):
            row0 = wid * _ROWS_PER_W + g * _G
            pltpu.sync_copy(
                x_hbm.at[pl.ds(row0 * NUM_INPUTS, _G * NUM_INPUTS)], xg_v)

            @plsc.parallel_loop(0, NUM_LUTS, step=1, unroll=8,
                                carry=jnp.zeros((_L,), jnp.float32))
            def body(l, acc):
                col = row_off + jnp.broadcast_to(l * ADDRESS_SIZE, (_L,))
                b0 = plsc.load_gather(xg_v, [col])
                b1 = plsc.load_gather(xg_v, [col + 1])
                b2 = plsc.load_gather(xg_v, [col + 2])
                b3 = plsc.load_gather(xg_v, [col + 3])
                addr = b0 + (b1 << 1) + (b2 << 2) + (b3 << 3)
                lut_idx = addr + jnp.broadcast_to(l * ENTRY_PER_LUT, (_L,))
                val = plsc.load_gather(luts_v, [lut_idx])
                return acc + val

            acc = body
            out_v[...] = acc
            pltpu.sync_copy(out_v, out_hbm.at[pl.ds(row0, _G)])

    return k


_kernel_call = _make_kernel()


@jax.jit
def kernel(x, luts):
    x32 = x.astype(jnp.int32)
    out = _kernel_call(x32.reshape(-1), luts.reshape(-1))
    return out.reshape(BATCH, 1)


# speedup vs baseline: 1.0522x; 1.0522x over previous
import functools

import jax
import jax.numpy as jnp
from jax import lax
from jax.experimental import pallas as pl
from jax.experimental.pallas import tpu as pltpu
from jax.experimental.pallas import tpu_sc as plsc

NUM_INPUTS = 4096
ADDRESS_SIZE = 4
NUM_LUTS = NUM_INPUTS // ADDRESS_SIZE
ENTRY_PER_LUT = 2 ** ADDRESS_SIZE
BATCH = 4096

_INFO = plsc.get_sparse_core_info()
_NC = _INFO.num_cores
_NS = _INFO.num_subcores
_L = _INFO.num_lanes
_NW = _NC * _NS
_ROWS_PER_W = BATCH // _NW
_G = 16
_NCHUNKS = _ROWS_PER_W // _G
_WORDS_PER_ROW = NUM_LUTS
_CHUNK_WORDS = _G * _WORDS_PER_ROW
_J = _WORDS_PER_ROW // _L


def _make_kernel():
    mesh = plsc.VectorSubcoreMesh(core_axis_name="c", subcore_axis_name="s")

    @functools.partial(
        pl.kernel,
        mesh=mesh,
        compiler_params=pltpu.CompilerParams(needs_layout_passes=False),
        out_type=jax.ShapeDtypeStruct((BATCH,), jnp.float32),
        scratch_types=[
            pltpu.VMEM((NUM_LUTS * ENTRY_PER_LUT,), jnp.float32),
            pltpu.VMEM((_CHUNK_WORDS,), jnp.int32),
            pltpu.VMEM((_CHUNK_WORDS,), jnp.int32),
            pltpu.VMEM((_G,), jnp.float32),
            pltpu.SemaphoreType.DMA,
            pltpu.SemaphoreType.DMA,
        ],
    )
    def k(xp_hbm, luts_hbm, out_hbm, luts_v, xa_v, xb_v, out_v, sa, sb):
        wid = lax.axis_index("s") * _NC + lax.axis_index("c")
        base = wid * _ROWS_PER_W * _WORDS_PER_ROW
        pltpu.sync_copy(luts_hbm, luts_v)
        lane = lax.broadcasted_iota(jnp.int32, (_L,), 0)
        lane16 = lane * ENTRY_PER_LUT

        bufs = (xa_v, xb_v)
        sems = (sa, sb)

        def start(g):
            return pltpu.async_copy(
                xp_hbm.at[pl.ds(base + g * _CHUNK_WORDS, _CHUNK_WORDS)],
                bufs[g % 2], sems[g % 2])

        pending = start(0)
        for g in range(_NCHUNKS):
            xg_v = bufs[g % 2]
            pending.wait()
            if g + 1 < _NCHUNKS:
                pending = start(g + 1)

            def row_body(r, outacc):
                rbase = r * _WORDS_PER_ROW

                @plsc.parallel_loop(0, _J, step=1, unroll=4,
                                    carry=jnp.zeros((_L,), jnp.float32))
                def acc(j, a):
                    v = xg_v[pl.ds(rbase + j * _L, _L)]
                    addr = (v + (v >> 7) + (v >> 14) + (v >> 21)) & 15
                    lut_idx = addr + lane16 + jnp.broadcast_to(
                        j * (_L * ENTRY_PER_LUT), (_L,))
                    return a + plsc.load_gather(luts_v, [lut_idx])

                s = jnp.sum(acc)
                return jnp.where(lane == r, s, outacc)

            outacc = lax.fori_loop(0, _G, row_body,
                                   jnp.zeros((_L,), jnp.float32))
            out_v[...] = outacc
            pltpu.sync_copy(
                out_v, out_hbm.at[pl.ds(wid * _ROWS_PER_W + g * _G, _G)])

    return k


_kernel_call = _make_kernel()


@jax.jit
def kernel(x, luts):
    x8 = x.astype(jnp.int8).reshape(BATCH, NUM_LUTS, ADDRESS_SIZE)
    xp = lax.bitcast_convert_type(x8, jnp.int32)
    out = _kernel_call(xp.reshape(-1), luts.reshape(-1))
    return out.reshape(BATCH, 1)


# speedup vs baseline: 4.8018x; 4.5638x over previous
import functools

import jax
import jax.numpy as jnp
from jax import lax
from jax.experimental import pallas as pl
from jax.experimental.pallas import tpu as pltpu
from jax.experimental.pallas import tpu_sc as plsc

NUM_INPUTS = 4096
ADDRESS_SIZE = 4
NUM_LUTS = NUM_INPUTS // ADDRESS_SIZE
ENTRY_PER_LUT = 2 ** ADDRESS_SIZE
BATCH = 4096

_INFO = plsc.get_sparse_core_info()
_NC = _INFO.num_cores
_NS = _INFO.num_subcores
_L = _INFO.num_lanes
_NW = _NC * _NS
_ROWS_PER_W = BATCH // _NW
_G = 16
_NCHUNKS = _ROWS_PER_W // _G
_NH = 2
_W = NUM_INPUTS // _NH
_LH = NUM_LUTS // _NH
_J = _LH // _L


def _make_kernel():
    mesh = plsc.VectorSubcoreMesh(core_axis_name="c", subcore_axis_name="s")

    @functools.partial(
        pl.kernel,
        mesh=mesh,
        compiler_params=pltpu.CompilerParams(needs_layout_passes=False),
        out_type=jax.ShapeDtypeStruct((BATCH,), jnp.float32),
        scratch_types=[
            pltpu.VMEM((ENTRY_PER_LUT * NUM_LUTS,), jnp.float32),
            pltpu.VMEM((_G, _W), jnp.int32),
            pltpu.VMEM((_G, _W), jnp.int32),
            pltpu.VMEM((_G,), jnp.float32),
            pltpu.SemaphoreType.DMA,
            pltpu.SemaphoreType.DMA,
        ],
    )
    def k(x_hbm, lutst_hbm, out_hbm, luts_v, xa_v, xb_v, out_v, sa, sb):
        wid = lax.axis_index("s") * _NC + lax.axis_index("c")
        row_base = wid * _ROWS_PER_W
        pltpu.sync_copy(lutst_hbm, luts_v)
        lane = lax.broadcasted_iota(jnp.int32, (_L,), 0)
        shiftv = [((lane >> 2) + i) & 3 for i in range(ADDRESS_SIZE)]
        patv = [lane * ADDRESS_SIZE + shiftv[i] for i in range(ADDRESS_SIZE)]

        bufs = (xa_v, xb_v)
        sems = (sa, sb)
        tasks = [(g, h) for g in range(_NCHUNKS) for h in range(_NH)]

        def start(t):
            g, h = tasks[t]
            return pltpu.async_copy(
                x_hbm.at[pl.ds(row_base + g * _G, _G), pl.ds(h * _W, _W)],
                bufs[t % 2], sems[t % 2])

        pending = start(0)
        for t, (g, h) in enumerate(tasks):
            xg_v = bufs[t % 2]
            pending.wait()
            if t + 1 < len(tasks):
                pending = start(t + 1)
            lbase = h * _LH

            if h == 0:
                out_v[...] = jnp.zeros((_L,), jnp.float32)

            @plsc.parallel_loop(0, _G, step=1, unroll=1)
            def _rows(r):
                rvec = jnp.broadcast_to(r, (_L,))

                @plsc.parallel_loop(0, _J, step=1, unroll=4,
                                    carry=jnp.zeros((_L,), jnp.float32))
                def acc(j, a):
                    cb = jnp.broadcast_to(j * (_L * ADDRESS_SIZE), (_L,))
                    addr = jnp.zeros((_L,), jnp.int32)
                    for i in range(ADDRESS_SIZE):
                        g_i = plsc.load_gather(xg_v, [rvec, cb + patv[i]])
                        addr = addr + (g_i << shiftv[i])
                    lidx = (addr << 10) + jnp.broadcast_to(
                        lbase + j * _L, (_L,)) + lane
                    return a + plsc.load_gather(luts_v, [lidx])

                s = jnp.sum(acc)
                cur = plsc.load_gather(out_v, [rvec])
                plsc.store_scatter(out_v, [rvec], cur + s,
                                   mask=(lane == 0))
            if h == _NH - 1:
                pltpu.sync_copy(
                    out_v, out_hbm.at[pl.ds(row_base + g * _G, _G)])

    return k


_kernel_call = _make_kernel()


@jax.jit
def kernel(x, luts):
    luts_t = luts.reshape(NUM_LUTS, ENTRY_PER_LUT).T.reshape(-1)
    out = _kernel_call(x, luts_t)
    return out.reshape(BATCH, 1)


# speedup vs baseline: 4.8549x; 1.0111x over previous
import functools

import jax
import jax.numpy as jnp
from jax import lax
from jax.experimental import pallas as pl
from jax.experimental.pallas import tpu as pltpu
from jax.experimental.pallas import tpu_sc as plsc

NUM_INPUTS = 4096
ADDRESS_SIZE = 4
NUM_LUTS = NUM_INPUTS // ADDRESS_SIZE
ENTRY_PER_LUT = 2 ** ADDRESS_SIZE
BATCH = 4096

_INFO = plsc.get_sparse_core_info()
_NC = _INFO.num_cores
_NS = _INFO.num_subcores
_L = _INFO.num_lanes
_NW = _NC * _NS
_ROWS_PER_W = BATCH // _NW
_G = 16
_NCHUNKS = _ROWS_PER_W // _G
_NH = 2
_W = NUM_INPUTS // _NH
_LH = NUM_LUTS // _NH
_J = _LH // _L


def _make_kernel():
    mesh = plsc.VectorSubcoreMesh(core_axis_name="c", subcore_axis_name="s")

    @functools.partial(
        pl.kernel,
        mesh=mesh,
        compiler_params=pltpu.CompilerParams(needs_layout_passes=False),
        out_type=jax.ShapeDtypeStruct((BATCH,), jnp.float32),
        scratch_types=[
            pltpu.VMEM((ENTRY_PER_LUT * NUM_LUTS,), jnp.float32),
            pltpu.VMEM((_G, _W), jnp.int32),
            pltpu.VMEM((_G, _W), jnp.int32),
            pltpu.VMEM((_G,), jnp.float32),
            pltpu.SemaphoreType.DMA,
            pltpu.SemaphoreType.DMA,
            pltpu.SemaphoreType.DMA,
        ],
    )
    def k(x_hbm, lutst_hbm, out_hbm, luts_v, xa_v, xb_v, out_v, sa, sb, sl):
        wid = lax.axis_index("s") * _NC + lax.axis_index("c")
        row_base = wid * _ROWS_PER_W
        luts_cpy = pltpu.async_copy(lutst_hbm, luts_v, sl)
        lane = lax.broadcasted_iota(jnp.int32, (_L,), 0)
        shiftv = [((lane >> 2) + i) & 3 for i in range(ADDRESS_SIZE)]
        patv = [lane * ADDRESS_SIZE + shiftv[i] for i in range(ADDRESS_SIZE)]

        bufs = (xa_v, xb_v)
        sems = (sa, sb)
        tasks = [(g, h) for g in range(_NCHUNKS) for h in range(_NH)]

        def start(t):
            g, h = tasks[t]
            return pltpu.async_copy(
                x_hbm.at[pl.ds(row_base + g * _G, _G), pl.ds(h * _W, _W)],
                bufs[t % 2], sems[t % 2])

        pending = start(0)
        luts_cpy.wait()
        for t, (g, h) in enumerate(tasks):
            xg_v = bufs[t % 2]
            pending.wait()
            if t + 1 < len(tasks):
                pending = start(t + 1)
            lbase = h * _LH

            if h == 0:
                out_v[...] = jnp.zeros((_L,), jnp.float32)

            @plsc.parallel_loop(0, _G, step=1, unroll=1)
            def _rows(r):
                rvec = jnp.broadcast_to(r, (_L,))

                @plsc.parallel_loop(0, _J, step=1, unroll=4,
                                    carry=jnp.zeros((_L,), jnp.float32))
                def acc(j, a):
                    cb = jnp.broadcast_to(j * (_L * ADDRESS_SIZE), (_L,))
                    addr = jnp.zeros((_L,), jnp.int32)
                    for i in range(ADDRESS_SIZE):
                        g_i = plsc.load_gather(xg_v, [rvec, cb + patv[i]])
                        addr = addr + (g_i << shiftv[i])
                    lidx = (addr << 10) + jnp.broadcast_to(
                        lbase + j * _L, (_L,)) + lane
                    return a + plsc.load_gather(luts_v, [lidx])

                s = jnp.sum(acc)
                cur = plsc.load_gather(out_v, [rvec])
                plsc.store_scatter(out_v, [rvec], cur + s,
                                   mask=(lane == 0))
            if h == _NH - 1:
                pltpu.sync_copy(
                    out_v, out_hbm.at[pl.ds(row_base + g * _G, _G)])

    return k


_kernel_call = _make_kernel()


@jax.jit
def kernel(x, luts):
    luts_t = luts.reshape(NUM_LUTS, ENTRY_PER_LUT).T.reshape(-1)
    out = _kernel_call(x, luts_t)
    return out.reshape(BATCH, 1)
